# Initial kernel scaffold; baseline (speedup 1.0000x reference)
import jax
import jax.numpy as jnp
from jax.experimental import pallas as pl


def kernel(pcd, mats, offset_2d, voxel_size_2d):
    raise NotImplementedError("write your pallas kernel here")


# speedup vs baseline: 3.4405x; 3.4405x over previous
import functools

import jax
import jax.numpy as jnp
from jax import lax
from jax.experimental import pallas as pl
from jax.experimental.pallas import tpu as pltpu
from jax.experimental.pallas import tpu_sc as plsc

GRID = 21
SIZE_2D = GRID * GRID
R = 8
B = 8
N = 500000
NW = 32
WPB = NW // B
PTS_W = N // WPB
GROUPS_W = PTS_W // 16
TAIL = PTS_W - GROUPS_W * 16
CHUNK_G = 434
CHUNK_P = CHUNK_G * 16
NCHUNK = GROUPS_W // CHUNK_G
LSTRIDE = 3536
HWORDS = 16 * LSTRIDE


def _hist_body(pcd_hbm, coef_hbm, out_hbm, buf, hist, partial, coefv):
    c = lax.axis_index("c")
    s = lax.axis_index("s")
    w = c * 16 + s
    b = w // WPB
    q = w % WPB
    woff = (b * N + q * PTS_W) * 3

    pltpu.sync_copy(coef_hbm, coefv)

    zeros = jnp.zeros((16,), jnp.float32)
    ones = jnp.ones((16,), jnp.float32)
    lane = lax.iota(jnp.int32, 16)
    lane_base = lane * LSTRIDE
    i3 = lane * 3

    def zbody(i, _):
        hist[pl.ds(i * 16, 16)] = zeros
        return 0
    lax.fori_loop(0, HWORDS // 16, zbody, 0)

    def do_group(xv, yv, mask=None):
        for r in range(R):
            base = r * 96
            axv = coefv[pl.ds(base, 16)]
            bxv = coefv[pl.ds(base + 16, 16)]
            cxv = coefv[pl.ds(base + 32, 16)]
            ayv = coefv[pl.ds(base + 48, 16)]
            byv = coefv[pl.ds(base + 64, 16)]
            cyv = coefv[pl.ds(base + 80, 16)]
            fx = xv * axv + yv * bxv + cxv
            fy = xv * ayv + yv * byv + cyv
            px = fx.astype(jnp.int32)
            py = fy.astype(jnp.int32)
            bidx = lane_base + (px * GRID + py + r * SIZE_2D)
            if mask is None:
                plsc.addupdate_scatter(hist, [bidx], ones)
            else:
                plsc.addupdate_scatter(hist, [bidx], ones, mask=mask)

    def chunk_body(ch, _):
        pltpu.sync_copy(
            pcd_hbm.at[pl.ds(woff + ch * (CHUNK_P * 3), CHUNK_P * 3)], buf)

        def gbody(g, _):
            gi = i3 + g * 48
            xv = plsc.load_gather(buf, [gi])
            yv = plsc.load_gather(buf, [gi + 1])
            do_group(xv, yv)
            return 0
        lax.fori_loop(0, CHUNK_G, gbody, 0)
        return 0
    lax.fori_loop(0, NCHUNK, chunk_body, 0)

    pltpu.sync_copy(
        pcd_hbm.at[pl.ds(woff + GROUPS_W * 48, TAIL * 3)],
        buf.at[pl.ds(0, TAIL * 3)])
    tmask = lane < TAIL
    xv = plsc.load_gather(buf, [i3], mask=tmask)
    yv = plsc.load_gather(buf, [i3 + 1], mask=tmask)
    do_group(xv, yv, mask=tmask)

    def rbody(jb, _):
        off = jb * 16
        v = hist[pl.ds(off, 16)]
        for i in range(1, 16):
            v = v + hist[pl.ds(i * LSTRIDE + off, 16)]
        partial[pl.ds(off, 16)] = v
        return 0
    lax.fori_loop(0, LSTRIDE // 16, rbody, 0)

    pltpu.sync_copy(partial, out_hbm.at[pl.ds(w * LSTRIDE, LSTRIDE)])


_hist_call = functools.partial(
    pl.kernel,
    mesh=plsc.VectorSubcoreMesh(core_axis_name="c", subcore_axis_name="s"),
    out_type=jax.ShapeDtypeStruct((NW * LSTRIDE,), jnp.float32),
    scratch_types=[
        pltpu.VMEM((CHUNK_P * 3,), jnp.float32),
        pltpu.VMEM((HWORDS,), jnp.float32),
        pltpu.VMEM((LSTRIDE,), jnp.float32),
        pltpu.VMEM((R * 6 * 16,), jnp.float32),
    ],
    compiler_params=pltpu.CompilerParams(needs_layout_passes=False),
)(_hist_body)


@jax.jit
def kernel(pcd, mats, offset_2d, voxel_size_2d):
    inv = (1.0 / voxel_size_2d).astype(jnp.float32)
    ax = mats[:, 0, 0] * inv[0]
    bx = mats[:, 0, 1] * inv[0]
    ay = mats[:, 1, 0] * inv[1]
    by = mats[:, 1, 1] * inv[1]
    cx = jnp.full((R,), offset_2d[0] * inv[0], jnp.float32)
    cy = jnp.full((R,), offset_2d[1] * inv[1], jnp.float32)
    coef = jnp.stack([ax, bx, cx, ay, by, cy], axis=1)
    coef16 = jnp.broadcast_to(
        coef[:, :, None], (R, 6, 16)).reshape(-1).astype(jnp.float32)
    pcd_flat = pcd.reshape(-1)

    out = _hist_call(pcd_flat, coef16)

    part = out.reshape(B, WPB, LSTRIDE).sum(axis=1)[:, : R * SIZE_2D]
    feat = part.reshape(B, R, SIZE_2D).transpose(0, 2, 1) / jnp.float32(N)
    return feat


# speedup vs baseline: 39.1583x; 11.3815x over previous
import functools

import jax
import jax.numpy as jnp
from jax import lax
from jax.experimental import pallas as pl
from jax.experimental.pallas import tpu as pltpu
from jax.experimental.pallas import tpu_sc as plsc

GRID = 21
SIZE_2D = GRID * GRID
R = 8
B = 8
N = 500000
NW = 32
BR = B * R * SIZE_2D
TILE = 128
TW = 122
WCOLS = TW * TILE
CW = 4096
CW_LAST = WCOLS - 3 * CW
REM0 = NW * WCOLS
TAIL0 = REM0 + 2 * TILE
TAILC = N - TAIL0


def _hist_body(pcd_hbm, tail_hbm, coef_hbm, out_hbm, xbuf, ybuf, hist, coefv,
               tbuf):
    c = lax.axis_index("c")
    s = lax.axis_index("s")
    w = c * 16 + s
    cstart_w = w * WCOLS

    pltpu.sync_copy(coef_hbm, coefv)

    zeros = jnp.zeros((16,), jnp.float32)
    ones = jnp.ones((16,), jnp.float32)
    lane = lax.iota(jnp.int32, 16)

    def zbody(i, _):
        hist[pl.ds(i * 16, 16)] = zeros
        return 0
    lax.fori_loop(0, BR // 16, zbody, 0)

    def do_group(xv, yv, boffv):
        for r in range(R):
            base = r * 96
            axv = coefv[pl.ds(base, 16)]
            bxv = coefv[pl.ds(base + 16, 16)]
            cxv = coefv[pl.ds(base + 32, 16)]
            ayv = coefv[pl.ds(base + 48, 16)]
            byv = coefv[pl.ds(base + 64, 16)]
            cyv = coefv[pl.ds(base + 80, 16)]
            fx = xv * axv + yv * bxv + cxv
            fy = xv * ayv + yv * byv + cyv
            px = fx.astype(jnp.int32)
            py = fy.astype(jnp.int32)
            bidx = boffv + (px * GRID + py + r * SIZE_2D)
            plsc.addupdate_scatter(hist, [bidx], ones)

    def process_block(cstart, cols):
        pltpu.sync_copy(pcd_hbm.at[0, :, pl.ds(cstart, cols)],
                        xbuf.at[:, pl.ds(0, cols)])
        pltpu.sync_copy(pcd_hbm.at[1, :, pl.ds(cstart, cols)],
                        ybuf.at[:, pl.ds(0, cols)])

        def rbody(row, _):
            rsplat = jnp.broadcast_to(row, (16,)).astype(jnp.int32)
            boffv = jnp.broadcast_to(row * (R * SIZE_2D), (16,)).astype(jnp.int32)

            def gbody(g, _):
                colv = lane + g * 16
                xv = plsc.load_gather(xbuf, [rsplat, colv])
                yv = plsc.load_gather(ybuf, [rsplat, colv])
                do_group(xv, yv, boffv)
                return 0
            lax.fori_loop(0, cols // 16, gbody, 0)
            return 0
        lax.fori_loop(0, B, rbody, 0)

    def chunk_body(i, _):
        process_block(cstart_w + i * CW, CW)
        return 0
    lax.fori_loop(0, 3, chunk_body, 0)
    process_block(cstart_w + 3 * CW, CW_LAST)

    @pl.when(w == 0)
    def _():
        process_block(REM0, TILE)

    @pl.when(w == 1)
    def _():
        process_block(REM0 + TILE, TILE)

    @pl.when(w == 2)
    def _():
        pltpu.sync_copy(tail_hbm, tbuf)

        def trbody(row, _):
            boffv = jnp.broadcast_to(row * (R * SIZE_2D), (16,)).astype(jnp.int32)
            for g in range(TAILC // 16):
                xv = tbuf[pl.ds(row * TAILC + g * 16, 16)]
                yv = tbuf[pl.ds(B * TAILC + row * TAILC + g * 16, 16)]
                do_group(xv, yv, boffv)
            return 0
        lax.fori_loop(0, B, trbody, 0)

    pltpu.sync_copy(hist, out_hbm.at[pl.ds(w * BR, BR)])


_hist_call = functools.partial(
    pl.kernel,
    mesh=plsc.VectorSubcoreMesh(core_axis_name="c", subcore_axis_name="s"),
    out_type=jax.ShapeDtypeStruct((NW * BR,), jnp.float32),
    scratch_types=[
        pltpu.VMEM((B, CW), jnp.float32),
        pltpu.VMEM((B, CW), jnp.float32),
        pltpu.VMEM((BR,), jnp.float32),
        pltpu.VMEM((R * 6 * 16,), jnp.float32),
        pltpu.VMEM((2 * B * TAILC,), jnp.float32),
    ],
    compiler_params=pltpu.CompilerParams(needs_layout_passes=False),
)(_hist_body)


@jax.jit
def kernel(pcd, mats, offset_2d, voxel_size_2d):
    inv = (1.0 / voxel_size_2d).astype(jnp.float32)
    ax = mats[:, 0, 0] * inv[0]
    bx = mats[:, 0, 1] * inv[0]
    ay = mats[:, 1, 0] * inv[1]
    by = mats[:, 1, 1] * inv[1]
    cx = jnp.full((R,), offset_2d[0] * inv[0], jnp.float32)
    cy = jnp.full((R,), offset_2d[1] * inv[1], jnp.float32)
    coef = jnp.stack([ax, bx, cx, ay, by, cy], axis=1)
    coef16 = jnp.broadcast_to(
        coef[:, :, None], (R, 6, 16)).reshape(-1).astype(jnp.float32)

    pcd_t = jnp.transpose(pcd, (2, 0, 1))
    tailxy = jnp.transpose(pcd[:, TAIL0:, :2], (2, 0, 1)).reshape(-1)

    out = _hist_call(pcd_t, tailxy, coef16)

    part = out.reshape(NW, B, R, SIZE_2D).sum(axis=0)
    feat = part.transpose(0, 2, 1) / jnp.float32(N)
    return feat


# speedup vs baseline: 122.3764x; 3.1252x over previous
import functools

import jax
import jax.numpy as jnp
from jax import lax
from jax.experimental import pallas as pl
from jax.experimental.pallas import tpu as pltpu
from jax.experimental.pallas import tpu_sc as plsc

GRID = 21
SIZE_2D = GRID * GRID
R = 8
B = 8
N = 500000
NW = 32
BR = B * R * SIZE_2D
TILE = 128
TW = 122
WCOLS = TW * TILE
CW = 4096
CW_LAST = WCOLS - 3 * CW
REM0 = NW * WCOLS
TAIL0 = REM0 + 2 * TILE
TAILC = N - TAIL0


def _hist_body(pcd_hbm, tail_hbm, coef_hbm, out_hbm, xbuf, ybuf, hist, coefv,
               tbuf):
    c = lax.axis_index("c")
    s = lax.axis_index("s")
    w = c * 16 + s
    cstart_w = w * WCOLS

    pltpu.sync_copy(coef_hbm, coefv)

    zeros = jnp.zeros((16,), jnp.float32)
    ones = jnp.ones((16,), jnp.float32)

    def zbody(i, _):
        hist[pl.ds(i * 16, 16)] = zeros
        return 0
    lax.fori_loop(0, BR // 16, zbody, 0)

    a0x = coefv[pl.ds(0, 16)]
    b0x = coefv[pl.ds(16, 16)]
    a0y = coefv[pl.ds(32, 16)]
    b0y = coefv[pl.ds(48, 16)]
    a1x = coefv[pl.ds(64, 16)]
    b1x = coefv[pl.ds(80, 16)]
    a1y = coefv[pl.ds(96, 16)]
    b1y = coefv[pl.ds(112, 16)]
    cxv = coefv[pl.ds(128, 16)]
    cyv = coefv[pl.ds(144, 16)]

    def do_group(xv, yv, boffs):
        u0 = xv * a0x + yv * b0x
        v0 = xv * a0y + yv * b0y
        u1 = xv * a1x + yv * b1x
        v1 = xv * a1y + yv * b1y
        fxy = (
            (u0, 1, v0, 1), (u1, 1, v1, 1),
            (v0, -1, u0, 1), (v1, -1, u1, 1),
            (u0, -1, v0, -1), (u1, -1, v1, -1),
            (v0, 1, u0, -1), (v1, 1, u1, -1),
        )
        for r, (ux, sx, uy, sy) in enumerate(fxy):
            fx = cxv + ux if sx > 0 else cxv - ux
            fy = cyv + uy if sy > 0 else cyv - uy
            px = fx.astype(jnp.int32)
            py = fy.astype(jnp.int32)
            bidx = px * GRID + py + boffs[r]
            plsc.addupdate_scatter(hist, [bidx], ones)

    def process_block(cstart, cols):
        pltpu.sync_copy(pcd_hbm.at[0, :, pl.ds(cstart, cols)],
                        xbuf.at[:, pl.ds(0, cols)])
        pltpu.sync_copy(pcd_hbm.at[1, :, pl.ds(cstart, cols)],
                        ybuf.at[:, pl.ds(0, cols)])

        def rbody(row, _):
            boffs = [
                jnp.broadcast_to(row * (R * SIZE_2D) + r * SIZE_2D,
                                 (16,)).astype(jnp.int32)
                for r in range(R)
            ]

            def gbody(g, _):
                xv = xbuf[row, pl.ds(g * 16, 16)]
                yv = ybuf[row, pl.ds(g * 16, 16)]
                do_group(xv, yv, boffs)
                return 0
            lax.fori_loop(0, cols // 16, gbody, 0)
            return 0
        lax.fori_loop(0, B, rbody, 0)

    def chunk_body(i, _):
        process_block(cstart_w + i * CW, CW)
        return 0
    lax.fori_loop(0, 3, chunk_body, 0)
    process_block(cstart_w + 3 * CW, CW_LAST)

    @pl.when(w == 0)
    def _():
        process_block(REM0, TILE)

    @pl.when(w == 1)
    def _():
        process_block(REM0 + TILE, TILE)

    @pl.when(w == 2)
    def _():
        pltpu.sync_copy(tail_hbm, tbuf)

        def trbody(row, _):
            boffs = [
                jnp.broadcast_to(row * (R * SIZE_2D) + r * SIZE_2D,
                                 (16,)).astype(jnp.int32)
                for r in range(R)
            ]
            for g in range(TAILC // 16):
                xv = tbuf[pl.ds(row * TAILC + g * 16, 16)]
                yv = tbuf[pl.ds(B * TAILC + row * TAILC + g * 16, 16)]
                do_group(xv, yv, boffs)
            return 0
        lax.fori_loop(0, B, trbody, 0)

    pltpu.sync_copy(hist, out_hbm.at[pl.ds(w * BR, BR)])


_hist_call = functools.partial(
    pl.kernel,
    mesh=plsc.VectorSubcoreMesh(core_axis_name="c", subcore_axis_name="s"),
    out_type=jax.ShapeDtypeStruct((NW * BR,), jnp.float32),
    scratch_types=[
        pltpu.VMEM((B, CW), jnp.float32),
        pltpu.VMEM((B, CW), jnp.float32),
        pltpu.VMEM((BR,), jnp.float32),
        pltpu.VMEM((10 * 16,), jnp.float32),
        pltpu.VMEM((2 * B * TAILC,), jnp.float32),
    ],
    compiler_params=pltpu.CompilerParams(needs_layout_passes=False),
)(_hist_body)


@jax.jit
def kernel(pcd, mats, offset_2d, voxel_size_2d):
    inv = (1.0 / voxel_size_2d).astype(jnp.float32)
    coef = jnp.stack([
        mats[0, 0, 0] * inv[0], mats[0, 0, 1] * inv[0],
        mats[0, 1, 0] * inv[1], mats[0, 1, 1] * inv[1],
        mats[1, 0, 0] * inv[0], mats[1, 0, 1] * inv[0],
        mats[1, 1, 0] * inv[1], mats[1, 1, 1] * inv[1],
        offset_2d[0] * inv[0], offset_2d[1] * inv[1],
    ])
    coef16 = jnp.broadcast_to(
        coef[:, None], (10, 16)).reshape(-1).astype(jnp.float32)

    pcd_t = jnp.transpose(pcd, (2, 0, 1))
    tailxy = jnp.transpose(pcd[:, TAIL0:, :2], (2, 0, 1)).reshape(-1)

    out = _hist_call(pcd_t, tailxy, coef16)

    part = out.reshape(NW, B, R, SIZE_2D).sum(axis=0)
    feat = part.transpose(0, 2, 1) / jnp.float32(N)
    return feat


# speedup vs baseline: 127.5589x; 1.0423x over previous
import functools

import jax
import jax.numpy as jnp
from jax import lax
from jax.experimental import pallas as pl
from jax.experimental.pallas import tpu as pltpu
from jax.experimental.pallas import tpu_sc as plsc

GRID = 21
SIZE_2D = GRID * GRID
R = 8
B = 8
N = 500000
NW = 32
BR = B * R * SIZE_2D
TILE = 128
TW = 122
WCOLS = TW * TILE
CW = 4096
CW_LAST = WCOLS - 3 * CW
REM0 = NW * WCOLS
TAIL0 = REM0 + 2 * TILE
TAILC = N - TAIL0


def _hist_body(pcd_hbm, tail_hbm, coef_hbm, out_hbm, xbuf, ybuf, hist, coefv,
               tbuf):
    c = lax.axis_index("c")
    s = lax.axis_index("s")
    w = c * 16 + s
    cstart_w = w * WCOLS

    pltpu.sync_copy(coef_hbm, coefv)

    zeros = jnp.zeros((16,), jnp.float32)
    ones = jnp.ones((16,), jnp.float32)

    def zbody(i, _):
        hist[pl.ds(i * 16, 16)] = zeros
        return 0
    lax.fori_loop(0, BR // 16, zbody, 0)

    a0x = coefv[pl.ds(0, 16)]
    b0x = coefv[pl.ds(16, 16)]
    a0y = coefv[pl.ds(32, 16)]
    b0y = coefv[pl.ds(48, 16)]
    a1x = coefv[pl.ds(64, 16)]
    b1x = coefv[pl.ds(80, 16)]
    a1y = coefv[pl.ds(96, 16)]
    b1y = coefv[pl.ds(112, 16)]
    cxv = coefv[pl.ds(128, 16)]
    cyv = coefv[pl.ds(144, 16)]

    def do_group(xv, yv, boffs):
        u0 = xv * a0x + yv * b0x
        v0 = xv * a0y + yv * b0y
        u1 = xv * a1x + yv * b1x
        v1 = xv * a1y + yv * b1y
        fxy = (
            (u0, 1, v0, 1), (u1, 1, v1, 1),
            (v0, -1, u0, 1), (v1, -1, u1, 1),
            (u0, -1, v0, -1), (u1, -1, v1, -1),
            (v0, 1, u0, -1), (v1, 1, u1, -1),
        )
        for r, (ux, sx, uy, sy) in enumerate(fxy):
            fx = cxv + ux if sx > 0 else cxv - ux
            fy = cyv + uy if sy > 0 else cyv - uy
            px = fx.astype(jnp.int32)
            py = fy.astype(jnp.int32)
            bidx = px * GRID + py + boffs[r]
            plsc.addupdate_scatter(hist, [bidx], ones)

    def process_block(cstart, cols):
        pltpu.sync_copy(pcd_hbm.at[0, :, pl.ds(cstart, cols)],
                        xbuf.at[:, pl.ds(0, cols)])
        pltpu.sync_copy(pcd_hbm.at[1, :, pl.ds(cstart, cols)],
                        ybuf.at[:, pl.ds(0, cols)])

        def rbody(row, _):
            boffs = [
                jnp.broadcast_to(row * (R * SIZE_2D) + r * SIZE_2D,
                                 (16,)).astype(jnp.int32)
                for r in range(R)
            ]

            def gbody(g4, _):
                base = g4 * 64
                for j in range(4):
                    xv = xbuf[row, pl.ds(base + j * 16, 16)]
                    yv = ybuf[row, pl.ds(base + j * 16, 16)]
                    do_group(xv, yv, boffs)
                return 0
            lax.fori_loop(0, cols // 64, gbody, 0)
            return 0
        lax.fori_loop(0, B, rbody, 0)

    def chunk_body(i, _):
        process_block(cstart_w + i * CW, CW)
        return 0
    lax.fori_loop(0, 3, chunk_body, 0)
    process_block(cstart_w + 3 * CW, CW_LAST)

    @pl.when(w == 0)
    def _():
        process_block(REM0, TILE)

    @pl.when(w == 1)
    def _():
        process_block(REM0 + TILE, TILE)

    @pl.when(w == 2)
    def _():
        pltpu.sync_copy(tail_hbm, tbuf)

        def trbody(row, _):
            boffs = [
                jnp.broadcast_to(row * (R * SIZE_2D) + r * SIZE_2D,
                                 (16,)).astype(jnp.int32)
                for r in range(R)
            ]
            for g in range(TAILC // 16):
                xv = tbuf[pl.ds(row * TAILC + g * 16, 16)]
                yv = tbuf[pl.ds(B * TAILC + row * TAILC + g * 16, 16)]
                do_group(xv, yv, boffs)
            return 0
        lax.fori_loop(0, B, trbody, 0)

    pltpu.sync_copy(hist, out_hbm.at[pl.ds(w * BR, BR)])


_hist_call = functools.partial(
    pl.kernel,
    mesh=plsc.VectorSubcoreMesh(core_axis_name="c", subcore_axis_name="s"),
    out_type=jax.ShapeDtypeStruct((NW * BR,), jnp.float32),
    scratch_types=[
        pltpu.VMEM((B, CW), jnp.float32),
        pltpu.VMEM((B, CW), jnp.float32),
        pltpu.VMEM((BR,), jnp.float32),
        pltpu.VMEM((10 * 16,), jnp.float32),
        pltpu.VMEM((2 * B * TAILC,), jnp.float32),
    ],
    compiler_params=pltpu.CompilerParams(needs_layout_passes=False),
)(_hist_body)


@jax.jit
def kernel(pcd, mats, offset_2d, voxel_size_2d):
    inv = (1.0 / voxel_size_2d).astype(jnp.float32)
    coef = jnp.stack([
        mats[0, 0, 0] * inv[0], mats[0, 0, 1] * inv[0],
        mats[0, 1, 0] * inv[1], mats[0, 1, 1] * inv[1],
        mats[1, 0, 0] * inv[0], mats[1, 0, 1] * inv[0],
        mats[1, 1, 0] * inv[1], mats[1, 1, 1] * inv[1],
        offset_2d[0] * inv[0], offset_2d[1] * inv[1],
    ])
    coef16 = jnp.broadcast_to(
        coef[:, None], (10, 16)).reshape(-1).astype(jnp.float32)

    pcd_t = jnp.transpose(pcd, (2, 0, 1))
    tailxy = jnp.transpose(pcd[:, TAIL0:, :2], (2, 0, 1)).reshape(-1)

    out = _hist_call(pcd_t, tailxy, coef16)

    part = out.reshape(NW, B, R, SIZE_2D).sum(axis=0)
    feat = part.transpose(0, 2, 1) / jnp.float32(N)
    return feat


# speedup vs baseline: 132.4245x; 1.0381x over previous
import functools

import jax
import jax.numpy as jnp
from jax import lax
from jax.experimental import pallas as pl
from jax.experimental.pallas import tpu as pltpu
from jax.experimental.pallas import tpu_sc as plsc

GRID = 21
SIZE_2D = GRID * GRID
R = 8
B = 8
N = 500000
NW = 32
RSTRIDE = 448
BR = B * R * RSTRIDE
TILE = 128
TW = 122
WCOLS = TW * TILE
CW = 4096
CW_LAST = WCOLS - 3 * CW
REM0 = NW * WCOLS
TAIL0 = REM0 + 2 * TILE
TAILC = N - TAIL0


def _hist_body(pcd_hbm, tail_hbm, coef_hbm, out_hbm, xbuf, ybuf, hist, coefv,
               tbuf):
    c = lax.axis_index("c")
    s = lax.axis_index("s")
    w = c * 16 + s
    cstart_w = w * WCOLS

    pltpu.sync_copy(coef_hbm, coefv)

    zeros = jnp.zeros((16,), jnp.float32)
    ones = jnp.ones((16,), jnp.float32)

    def zbody(i, _):
        hist[pl.ds(i * 16, 16)] = zeros
        return 0
    lax.fori_loop(0, BR // 16, zbody, 0)

    a0x = coefv[pl.ds(0, 16)]
    b0x = coefv[pl.ds(16, 16)]
    a0y = coefv[pl.ds(32, 16)]
    b0y = coefv[pl.ds(48, 16)]
    a1x = coefv[pl.ds(64, 16)]
    b1x = coefv[pl.ds(80, 16)]
    a1y = coefv[pl.ds(96, 16)]
    b1y = coefv[pl.ds(112, 16)]
    cxv = coefv[pl.ds(128, 16)]
    cyv = coefv[pl.ds(144, 16)]

    def do_group(xv, yv, boffs):
        u0 = xv * a0x + yv * b0x
        v0 = xv * a0y + yv * b0y
        u1 = xv * a1x + yv * b1x
        v1 = xv * a1y + yv * b1y
        fxy = (
            (u0, 1, v0, 1), (u1, 1, v1, 1),
            (v0, -1, u0, 1), (v1, -1, u1, 1),
            (u0, -1, v0, -1), (u1, -1, v1, -1),
            (v0, 1, u0, -1), (v1, 1, u1, -1),
        )
        for r, (ux, sx, uy, sy) in enumerate(fxy):
            fx = cxv + ux if sx > 0 else cxv - ux
            fy = cyv + uy if sy > 0 else cyv - uy
            px = fx.astype(jnp.int32)
            py = fy.astype(jnp.int32)
            bidx = px * GRID + py
            plsc.addupdate_scatter(
                hist.at[pl.ds(boffs + r * RSTRIDE, RSTRIDE)], [bidx], ones)

    def process_block(cstart, cols):
        pltpu.sync_copy(pcd_hbm.at[0, :, pl.ds(cstart, cols)],
                        xbuf.at[:, pl.ds(0, cols)])
        pltpu.sync_copy(pcd_hbm.at[1, :, pl.ds(cstart, cols)],
                        ybuf.at[:, pl.ds(0, cols)])

        def rbody(row, _):
            boffs = row * (R * RSTRIDE)

            def gbody(g4, _):
                base = g4 * 64
                for j in range(4):
                    xv = xbuf[row, pl.ds(base + j * 16, 16)]
                    yv = ybuf[row, pl.ds(base + j * 16, 16)]
                    do_group(xv, yv, boffs)
                return 0
            lax.fori_loop(0, cols // 64, gbody, 0)
            return 0
        lax.fori_loop(0, B, rbody, 0)

    def chunk_body(i, _):
        process_block(cstart_w + i * CW, CW)
        return 0
    lax.fori_loop(0, 3, chunk_body, 0)
    process_block(cstart_w + 3 * CW, CW_LAST)

    @pl.when(w == 0)
    def _():
        process_block(REM0, TILE)

    @pl.when(w == 1)
    def _():
        process_block(REM0 + TILE, TILE)

    @pl.when(w == 2)
    def _():
        pltpu.sync_copy(tail_hbm, tbuf)

        def trbody(row, _):
            boffs = row * (R * RSTRIDE)
            for g in range(TAILC // 16):
                xv = tbuf[pl.ds(row * TAILC + g * 16, 16)]
                yv = tbuf[pl.ds(B * TAILC + row * TAILC + g * 16, 16)]
                do_group(xv, yv, boffs)
            return 0
        lax.fori_loop(0, B, trbody, 0)

    pltpu.sync_copy(hist, out_hbm.at[pl.ds(w * BR, BR)])


_hist_call = functools.partial(
    pl.kernel,
    mesh=plsc.VectorSubcoreMesh(core_axis_name="c", subcore_axis_name="s"),
    out_type=jax.ShapeDtypeStruct((NW * BR,), jnp.float32),
    scratch_types=[
        pltpu.VMEM((B, CW), jnp.float32),
        pltpu.VMEM((B, CW), jnp.float32),
        pltpu.VMEM((BR,), jnp.float32),
        pltpu.VMEM((10 * 16,), jnp.float32),
        pltpu.VMEM((2 * B * TAILC,), jnp.float32),
    ],
    compiler_params=pltpu.CompilerParams(needs_layout_passes=False),
)(_hist_body)


@jax.jit
def kernel(pcd, mats, offset_2d, voxel_size_2d):
    inv = (1.0 / voxel_size_2d).astype(jnp.float32)
    coef = jnp.stack([
        mats[0, 0, 0] * inv[0], mats[0, 0, 1] * inv[0],
        mats[0, 1, 0] * inv[1], mats[0, 1, 1] * inv[1],
        mats[1, 0, 0] * inv[0], mats[1, 0, 1] * inv[0],
        mats[1, 1, 0] * inv[1], mats[1, 1, 1] * inv[1],
        offset_2d[0] * inv[0], offset_2d[1] * inv[1],
    ])
    coef16 = jnp.broadcast_to(
        coef[:, None], (10, 16)).reshape(-1).astype(jnp.float32)

    pcd_t = jnp.transpose(pcd, (2, 0, 1))
    tailxy = jnp.transpose(pcd[:, TAIL0:, :2], (2, 0, 1)).reshape(-1)

    out = _hist_call(pcd_t, tailxy, coef16)

    part = out.reshape(NW, B, R, RSTRIDE)[..., :SIZE_2D].sum(axis=0)
    feat = part.transpose(0, 2, 1) / jnp.float32(N)
    return feat


# speedup vs baseline: 133.0660x; 1.0048x over previous
import functools

import jax
import jax.numpy as jnp
from jax import lax
from jax.experimental import pallas as pl
from jax.experimental.pallas import tpu as pltpu
from jax.experimental.pallas import tpu_sc as plsc

GRID = 21
SIZE_2D = GRID * GRID
R = 8
B = 8
N = 500000
NW = 32
RSTRIDE = 448
BR = B * R * RSTRIDE
TILE = 128
TW = 122
WCOLS = TW * TILE
CW = 4096
CW_LAST = WCOLS - 3 * CW
REM0 = NW * WCOLS
TAIL0 = REM0 + 2 * TILE
TAILC = N - TAIL0


def _hist_body(pcd_hbm, tail_hbm, coef_hbm, out_hbm, xbuf, ybuf, hist, coefv,
               tbuf, dsem):
    c = lax.axis_index("c")
    s = lax.axis_index("s")
    w = c * 16 + s
    cstart_w = w * WCOLS

    pltpu.sync_copy(coef_hbm, coefv)

    zeros = jnp.zeros((16,), jnp.float32)
    ones = jnp.ones((16,), jnp.float32)

    def zbody(i, _):
        hist[pl.ds(i * 16, 16)] = zeros
        return 0
    lax.fori_loop(0, BR // 16, zbody, 0)

    a0x = coefv[pl.ds(0, 16)]
    b0x = coefv[pl.ds(16, 16)]
    a0y = coefv[pl.ds(32, 16)]
    b0y = coefv[pl.ds(48, 16)]
    a1x = coefv[pl.ds(64, 16)]
    b1x = coefv[pl.ds(80, 16)]
    a1y = coefv[pl.ds(96, 16)]
    b1y = coefv[pl.ds(112, 16)]
    cxv = coefv[pl.ds(128, 16)]
    cyv = coefv[pl.ds(144, 16)]

    def do_group(xv, yv, boffs):
        u0 = xv * a0x + yv * b0x
        v0 = xv * a0y + yv * b0y
        u1 = xv * a1x + yv * b1x
        v1 = xv * a1y + yv * b1y
        fxy = (
            (u0, 1, v0, 1), (u1, 1, v1, 1),
            (v0, -1, u0, 1), (v1, -1, u1, 1),
            (u0, -1, v0, -1), (u1, -1, v1, -1),
            (v0, 1, u0, -1), (v1, 1, u1, -1),
        )
        for r, (ux, sx, uy, sy) in enumerate(fxy):
            fx = cxv + ux if sx > 0 else cxv - ux
            fy = cyv + uy if sy > 0 else cyv - uy
            px = fx.astype(jnp.int32)
            py = fy.astype(jnp.int32)
            bidx = px * GRID + py
            plsc.addupdate_scatter(
                hist.at[pl.ds(boffs + r * RSTRIDE, RSTRIDE)], [bidx], ones)

    def process_block(cstart, cols, sem):
        cpx = pltpu.async_copy(pcd_hbm.at[0, :, pl.ds(cstart, cols)],
                               xbuf.at[:, pl.ds(0, cols)], sem)
        cpy = pltpu.async_copy(pcd_hbm.at[1, :, pl.ds(cstart, cols)],
                               ybuf.at[:, pl.ds(0, cols)], sem)
        cpx.wait()
        cpy.wait()

        def rbody(row, _):
            boffs = row * (R * RSTRIDE)

            def gbody(g8, _):
                base = g8 * 128
                for j in range(8):
                    xv = xbuf[row, pl.ds(base + j * 16, 16)]
                    yv = ybuf[row, pl.ds(base + j * 16, 16)]
                    do_group(xv, yv, boffs)
                return 0
            lax.fori_loop(0, cols // 128, gbody, 0)
            return 0
        lax.fori_loop(0, B, rbody, 0)

    def chunk_body(i, _):
        process_block(cstart_w + i * CW, CW, dsem)
        return 0
    lax.fori_loop(0, 3, chunk_body, 0)
    process_block(cstart_w + 3 * CW, CW_LAST, dsem)

    @pl.when(w == 0)
    def _():
        process_block(REM0, TILE, dsem)

    @pl.when(w == 1)
    def _():
        process_block(REM0 + TILE, TILE, dsem)

    @pl.when(w == 2)
    def _():
        pltpu.sync_copy(tail_hbm, tbuf)

        def trbody(row, _):
            boffs = row * (R * RSTRIDE)
            for g in range(TAILC // 16):
                xv = tbuf[pl.ds(row * TAILC + g * 16, 16)]
                yv = tbuf[pl.ds(B * TAILC + row * TAILC + g * 16, 16)]
                do_group(xv, yv, boffs)
            return 0
        lax.fori_loop(0, B, trbody, 0)

    pltpu.sync_copy(hist, out_hbm.at[pl.ds(w * BR, BR)])


_hist_call = functools.partial(
    pl.kernel,
    mesh=plsc.VectorSubcoreMesh(core_axis_name="c", subcore_axis_name="s"),
    out_type=jax.ShapeDtypeStruct((NW * BR,), jnp.float32),
    scratch_types=[
        pltpu.VMEM((B, CW), jnp.float32),
        pltpu.VMEM((B, CW), jnp.float32),
        pltpu.VMEM((BR,), jnp.float32),
        pltpu.VMEM((10 * 16,), jnp.float32),
        pltpu.VMEM((2 * B * TAILC,), jnp.float32),
        pltpu.SemaphoreType.DMA,
    ],
    compiler_params=pltpu.CompilerParams(needs_layout_passes=False),
)(_hist_body)


@jax.jit
def kernel(pcd, mats, offset_2d, voxel_size_2d):
    inv = (1.0 / voxel_size_2d).astype(jnp.float32)
    coef = jnp.stack([
        mats[0, 0, 0] * inv[0], mats[0, 0, 1] * inv[0],
        mats[0, 1, 0] * inv[1], mats[0, 1, 1] * inv[1],
        mats[1, 0, 0] * inv[0], mats[1, 0, 1] * inv[0],
        mats[1, 1, 0] * inv[1], mats[1, 1, 1] * inv[1],
        offset_2d[0] * inv[0], offset_2d[1] * inv[1],
    ])
    coef16 = jnp.broadcast_to(
        coef[:, None], (10, 16)).reshape(-1).astype(jnp.float32)

    pcd_t = jnp.transpose(pcd, (2, 0, 1))
    tailxy = jnp.transpose(pcd[:, TAIL0:, :2], (2, 0, 1)).reshape(-1)

    out = _hist_call(pcd_t, tailxy, coef16)

    part = out.reshape(NW, B, R, RSTRIDE)[..., :SIZE_2D].sum(axis=0)
    feat = part.transpose(0, 2, 1) / jnp.float32(N)
    return feat


# speedup vs baseline: 151.9785x; 1.1421x over previous
import functools

import jax
import jax.numpy as jnp
from jax import lax
from jax.experimental import pallas as pl
from jax.experimental.pallas import tpu as pltpu
from jax.experimental.pallas import tpu_sc as plsc

GRID = 21
SIZE_2D = GRID * GRID
R = 8
B = 8
N = 500000
NW = 32
RSTRIDE = 448
BR = B * R * RSTRIDE
TILE = 128
TW = 122
WCOLS = TW * TILE
CW = 4096
CW_LAST = WCOLS - 3 * CW
REM0 = NW * WCOLS
TAIL0 = REM0 + 2 * TILE
TAILC = N - TAIL0


def _hist_body(pcd_hbm, tail_hbm, coef_hbm, out_hbm, xbuf, ybuf, hist, coefv,
               tbuf, dsem):
    c = lax.axis_index("c")
    s = lax.axis_index("s")
    w = c * 16 + s
    cstart_w = w * WCOLS

    pltpu.sync_copy(coef_hbm, coefv)

    zeros = jnp.zeros((16,), jnp.float32)
    ones = jnp.ones((16,), jnp.float32)

    def zbody(i, _):
        hist[pl.ds(i * 16, 16)] = zeros
        return 0
    lax.fori_loop(0, BR // 16, zbody, 0)

    a0x = coefv[pl.ds(0, 16)]
    b0x = coefv[pl.ds(16, 16)]
    a0y = coefv[pl.ds(32, 16)]
    b0y = coefv[pl.ds(48, 16)]
    a1x = coefv[pl.ds(64, 16)]
    b1x = coefv[pl.ds(80, 16)]
    a1y = coefv[pl.ds(96, 16)]
    b1y = coefv[pl.ds(112, 16)]
    cxv = coefv[pl.ds(128, 16)]
    cyv = coefv[pl.ds(144, 16)]

    def do_group(xv, yv, boffs):
        u0 = xv * a0x + yv * b0x
        v0 = xv * a0y + yv * b0y
        u1 = xv * a1x + yv * b1x
        v1 = xv * a1y + yv * b1y
        pa0 = (cxv + u0).astype(jnp.int32)
        pb0 = (cxv + v0).astype(jnp.int32)
        pc0 = (cxv - u0).astype(jnp.int32)
        pd0 = (cxv - v0).astype(jnp.int32)
        pa1 = (cxv + u1).astype(jnp.int32)
        pb1 = (cxv + v1).astype(jnp.int32)
        pc1 = (cxv - u1).astype(jnp.int32)
        pd1 = (cxv - v1).astype(jnp.int32)
        pairs = (
            (pa0, pb0), (pa1, pb1),
            (pd0, pa0), (pd1, pa1),
            (pc0, pd0), (pc1, pd1),
            (pb0, pc0), (pb1, pc1),
        )
        for r, (px, py) in enumerate(pairs):
            plsc.addupdate_scatter(
                hist.at[pl.ds(boffs + r * RSTRIDE, RSTRIDE)],
                [px * GRID + py], ones)

    def process_block(cstart, cols, sem):
        cpx = pltpu.async_copy(pcd_hbm.at[0, :, pl.ds(cstart, cols)],
                               xbuf.at[:, pl.ds(0, cols)], sem)
        cpy = pltpu.async_copy(pcd_hbm.at[1, :, pl.ds(cstart, cols)],
                               ybuf.at[:, pl.ds(0, cols)], sem)
        cpx.wait()
        cpy.wait()

        def rbody(row, _):
            boffs = row * (R * RSTRIDE)

            def gbody(g8, _):
                base = g8 * 128
                for j in range(8):
                    xv = xbuf[row, pl.ds(base + j * 16, 16)]
                    yv = ybuf[row, pl.ds(base + j * 16, 16)]
                    do_group(xv, yv, boffs)
                return 0
            lax.fori_loop(0, cols // 128, gbody, 0)
            return 0
        lax.fori_loop(0, B, rbody, 0)

    def chunk_body(i, _):
        process_block(cstart_w + i * CW, CW, dsem)
        return 0
    lax.fori_loop(0, 3, chunk_body, 0)
    process_block(cstart_w + 3 * CW, CW_LAST, dsem)

    @pl.when(w == 0)
    def _():
        process_block(REM0, TILE, dsem)

    @pl.when(w == 1)
    def _():
        process_block(REM0 + TILE, TILE, dsem)

    @pl.when(w == 2)
    def _():
        pltpu.sync_copy(tail_hbm, tbuf)

        def trbody(row, _):
            boffs = row * (R * RSTRIDE)
            for g in range(TAILC // 16):
                xv = tbuf[pl.ds(row * TAILC + g * 16, 16)]
                yv = tbuf[pl.ds(B * TAILC + row * TAILC + g * 16, 16)]
                do_group(xv, yv, boffs)
            return 0
        lax.fori_loop(0, B, trbody, 0)

    pltpu.sync_copy(hist, out_hbm.at[pl.ds(w * BR, BR)])


_hist_call = functools.partial(
    pl.kernel,
    mesh=plsc.VectorSubcoreMesh(core_axis_name="c", subcore_axis_name="s"),
    out_type=jax.ShapeDtypeStruct((NW * BR,), jnp.float32),
    scratch_types=[
        pltpu.VMEM((B, CW), jnp.float32),
        pltpu.VMEM((B, CW), jnp.float32),
        pltpu.VMEM((BR,), jnp.float32),
        pltpu.VMEM((10 * 16,), jnp.float32),
        pltpu.VMEM((2 * B * TAILC,), jnp.float32),
        pltpu.SemaphoreType.DMA,
    ],
    compiler_params=pltpu.CompilerParams(needs_layout_passes=False),
)(_hist_body)


@jax.jit
def kernel(pcd, mats, offset_2d, voxel_size_2d):
    inv = (1.0 / voxel_size_2d).astype(jnp.float32)
    coef = jnp.stack([
        mats[0, 0, 0] * inv[0], mats[0, 0, 1] * inv[0],
        mats[0, 1, 0] * inv[1], mats[0, 1, 1] * inv[1],
        mats[1, 0, 0] * inv[0], mats[1, 0, 1] * inv[0],
        mats[1, 1, 0] * inv[1], mats[1, 1, 1] * inv[1],
        offset_2d[0] * inv[0], offset_2d[1] * inv[1],
    ])
    coef16 = jnp.broadcast_to(
        coef[:, None], (10, 16)).reshape(-1).astype(jnp.float32)

    pcd_t = jnp.transpose(pcd, (2, 0, 1))
    tailxy = jnp.transpose(pcd[:, TAIL0:, :2], (2, 0, 1)).reshape(-1)

    out = _hist_call(pcd_t, tailxy, coef16)

    part = out.reshape(NW, B, R, RSTRIDE)[..., :SIZE_2D].sum(axis=0)
    feat = part.transpose(0, 2, 1) / jnp.float32(N)
    return feat


# speedup vs baseline: 194.4483x; 1.2794x over previous
import functools

import jax
import jax.numpy as jnp
from jax import lax
from jax.experimental import pallas as pl
from jax.experimental.pallas import tpu as pltpu
from jax.experimental.pallas import tpu_sc as plsc

GRID = 21
SIZE_2D = GRID * GRID
R = 8
B = 8
N = 500000
NW = 32
RSTRIDE = 448
BR = B * R * RSTRIDE
TILE = 128
TW = 122
WCOLS = TW * TILE
CW = 4096
CW_LAST = WCOLS - 3 * CW
REM0 = NW * WCOLS
TAIL0 = REM0 + 2 * TILE
TAILC = N - TAIL0


def _hist_body(pcd_hbm, tail_hbm, coef_hbm, out_hbm, xbuf, ybuf, hist, coefv,
               tbuf, dsem):
    c = lax.axis_index("c")
    s = lax.axis_index("s")
    w = c * 16 + s
    cstart_w = w * WCOLS

    pltpu.sync_copy(coef_hbm, coefv)

    zeros = jnp.zeros((16,), jnp.int32)
    ones = jnp.ones((16,), jnp.int32)

    def zbody(i, _):
        hist[pl.ds(i * 16, 16)] = zeros
        return 0
    lax.fori_loop(0, BR // 16, zbody, 0)

    a0x = coefv[pl.ds(0, 16)]
    b0x = coefv[pl.ds(16, 16)]
    a0y = coefv[pl.ds(32, 16)]
    b0y = coefv[pl.ds(48, 16)]
    a1x = coefv[pl.ds(64, 16)]
    b1x = coefv[pl.ds(80, 16)]
    a1y = coefv[pl.ds(96, 16)]
    b1y = coefv[pl.ds(112, 16)]
    cxv = coefv[pl.ds(128, 16)]
    cyv = coefv[pl.ds(144, 16)]

    def do_group(xv, yv, boffs):
        u0 = xv * a0x + yv * b0x
        v0 = xv * a0y + yv * b0y
        u1 = xv * a1x + yv * b1x
        v1 = xv * a1y + yv * b1y
        pa0 = (cxv + u0).astype(jnp.int32)
        pb0 = (cxv + v0).astype(jnp.int32)
        pc0 = (cxv - u0).astype(jnp.int32)
        pd0 = (cxv - v0).astype(jnp.int32)
        pa1 = (cxv + u1).astype(jnp.int32)
        pb1 = (cxv + v1).astype(jnp.int32)
        pc1 = (cxv - u1).astype(jnp.int32)
        pd1 = (cxv - v1).astype(jnp.int32)
        pairs = (
            (pa0, pb0), (pa1, pb1),
            (pd0, pa0), (pd1, pa1),
            (pc0, pd0), (pc1, pd1),
            (pb0, pc0), (pb1, pc1),
        )
        for r, (px, py) in enumerate(pairs):
            plsc.addupdate_scatter(
                hist.at[pl.ds(boffs + r * RSTRIDE, RSTRIDE)],
                [px * GRID + py], ones)

    def process_block(cstart, cols, sem):
        cpx = pltpu.async_copy(pcd_hbm.at[0, :, pl.ds(cstart, cols)],
                               xbuf.at[:, pl.ds(0, cols)], sem)
        cpy = pltpu.async_copy(pcd_hbm.at[1, :, pl.ds(cstart, cols)],
                               ybuf.at[:, pl.ds(0, cols)], sem)
        cpx.wait()
        cpy.wait()

        def rbody(row, _):
            boffs = row * (R * RSTRIDE)

            def gbody(g8, _):
                base = g8 * 128
                for j in range(8):
                    xv = xbuf[row, pl.ds(base + j * 16, 16)]
                    yv = ybuf[row, pl.ds(base + j * 16, 16)]
                    do_group(xv, yv, boffs)
                return 0
            lax.fori_loop(0, cols // 128, gbody, 0)
            return 0
        lax.fori_loop(0, B, rbody, 0)

    def chunk_body(i, _):
        process_block(cstart_w + i * CW, CW, dsem)
        return 0
    lax.fori_loop(0, 3, chunk_body, 0)
    process_block(cstart_w + 3 * CW, CW_LAST, dsem)

    @pl.when(w == 0)
    def _():
        process_block(REM0, TILE, dsem)

    @pl.when(w == 1)
    def _():
        process_block(REM0 + TILE, TILE, dsem)

    @pl.when(w == 2)
    def _():
        pltpu.sync_copy(tail_hbm, tbuf)

        def trbody(row, _):
            boffs = row * (R * RSTRIDE)
            for g in range(TAILC // 16):
                xv = tbuf[pl.ds(row * TAILC + g * 16, 16)]
                yv = tbuf[pl.ds(B * TAILC + row * TAILC + g * 16, 16)]
                do_group(xv, yv, boffs)
            return 0
        lax.fori_loop(0, B, trbody, 0)

    pltpu.sync_copy(hist, out_hbm.at[pl.ds(w * BR, BR)])


_hist_call = functools.partial(
    pl.kernel,
    mesh=plsc.VectorSubcoreMesh(core_axis_name="c", subcore_axis_name="s"),
    out_type=jax.ShapeDtypeStruct((NW * BR,), jnp.int32),
    scratch_types=[
        pltpu.VMEM((B, CW), jnp.float32),
        pltpu.VMEM((B, CW), jnp.float32),
        pltpu.VMEM((BR,), jnp.int32),
        pltpu.VMEM((10 * 16,), jnp.float32),
        pltpu.VMEM((2 * B * TAILC,), jnp.float32),
        pltpu.SemaphoreType.DMA,
    ],
    compiler_params=pltpu.CompilerParams(needs_layout_passes=False),
)(_hist_body)


@jax.jit
def kernel(pcd, mats, offset_2d, voxel_size_2d):
    inv = (1.0 / voxel_size_2d).astype(jnp.float32)
    coef = jnp.stack([
        mats[0, 0, 0] * inv[0], mats[0, 0, 1] * inv[0],
        mats[0, 1, 0] * inv[1], mats[0, 1, 1] * inv[1],
        mats[1, 0, 0] * inv[0], mats[1, 0, 1] * inv[0],
        mats[1, 1, 0] * inv[1], mats[1, 1, 1] * inv[1],
        offset_2d[0] * inv[0], offset_2d[1] * inv[1],
    ])
    coef16 = jnp.broadcast_to(
        coef[:, None], (10, 16)).reshape(-1).astype(jnp.float32)

    pcd_t = jnp.transpose(pcd, (2, 0, 1))
    tailxy = jnp.transpose(pcd[:, TAIL0:, :2], (2, 0, 1)).reshape(-1)

    out = _hist_call(pcd_t, tailxy, coef16)

    part = out.reshape(NW, B, R, RSTRIDE)[..., :SIZE_2D].sum(axis=0).astype(jnp.float32)
    feat = part.transpose(0, 2, 1) / jnp.float32(N)
    return feat
